# Initial kernel scaffold; baseline (speedup 1.0000x reference)
import jax
import jax.numpy as jnp
from jax.experimental import pallas as pl


def kernel(x, edge_index, edge_attr, basis, att, root, bias):
    raise NotImplementedError("write your pallas kernel here")


# speedup vs baseline: 2.9863x; 2.9863x over previous
import functools

import jax
import jax.numpy as jnp
from jax import lax
from jax.experimental import pallas as pl
from jax.experimental.pallas import tpu as pltpu
from jax.experimental.pallas import tpu_sc as plsc

N = 10000
E = 160000
IN = 128
OUT = 128
R = 4
NB = 2

NC = 2
NS = 16
L = 16
NW = NC * NS
CH = 64
EPT = 5120
NCH = EPT // CH
UNROLL = 4
E_PAD = NW * EPT
ROWS_PT = 624
REM_ROWS = N - NS * ROWS_PT


def _tc_prep(x_ref, basis_ref, root_ref, bias_ref, eat_ref, att_ref,
             y_ref, base_ref, c0_ref, c1_ref):
    xx = x_ref[...]
    y_ref[:, :OUT] = jnp.dot(xx, basis_ref[0], preferred_element_type=jnp.float32)
    y_ref[:, OUT:] = jnp.dot(xx, basis_ref[1], preferred_element_type=jnp.float32)
    base_ref[...] = (jnp.dot(xx, root_ref[...], preferred_element_type=jnp.float32)
                     + bias_ref[...][None, :])
    ea = eat_ref[...]
    att = att_ref[...]
    c0_ref[...] = jnp.sum(ea * att[:, 0][:, None], axis=0)
    c1_ref[...] = jnp.sum(ea * att[:, 1][:, None], axis=0)


def _tc_final(base_ref, part_ref, o_ref):
    o_ref[...] = base_ref[...] + part_ref[0] + part_ref[1]


def _splat(v, k):
    idx = jnp.full((L, 1), k, jnp.int32)
    dnums = lax.GatherDimensionNumbers(
        offset_dims=(), collapsed_slice_dims=(0,), start_index_map=(0,))
    return lax.gather(v, idx, dnums, (1,),
                      mode=lax.GatherScatterMode.PROMISE_IN_BOUNDS)


def _sc_edges(y_hbm, idx_hbm, cf_hbm, out_hbm,
              ibuf, cbuf, rows0, rows1, msgv, accum,
              semi0, semi1, semi2, semi3,
              semc0, semc1, semc2, semc3, semg0, semg1):
    cid = lax.axis_index("c")
    sid = lax.axis_index("s")
    t = cid * NS + sid
    rows = (rows0, rows1)
    semg = (semg0, semg1)
    semi = (semi0, semi1, semi2, semi3)
    semc = (semc0, semc1, semc2, semc3)

    def _zrow(i, carry):
        for h in range(OUT // L):
            msgv[i, pl.ds(h * L, L)] = jnp.zeros((L,), jnp.float32)
        return carry
    lax.fori_loop(0, CH, _zrow, 0)

    base_row = sid * ROWS_PT
    nfull = ROWS_PT // CH
    rem = ROWS_PT - nfull * CH
    for i in range(nfull):
        pltpu.sync_copy(msgv, accum.at[pl.ds(base_row + i * CH, CH)])
    if rem:
        pltpu.sync_copy(msgv.at[pl.ds(0, rem)],
                        accum.at[pl.ds(base_row + nfull * CH, rem)])

    @pl.when(sid == NS - 1)
    def _zero_tail():
        pltpu.sync_copy(msgv.at[pl.ds(0, REM_ROWS)],
                        accum.at[pl.ds(NS * ROWS_PT, REM_ROWS)])
    plsc.subcore_barrier()

    def i_copy(jj, r):
        return pltpu.make_async_copy(idx_hbm.at[t, jj], ibuf.at[r], semi[r])

    def c_copy(jj, r):
        return pltpu.make_async_copy(cf_hbm.at[t, jj], cbuf.at[r], semc[r])

    def g_copy(jj, r, b):
        return pltpu.make_async_copy(y_hbm.at[ibuf.at[r, 0]], rows[b], semg[b])

    def chunk(jj, r, b):
        @pl.when(jj + 1 < NCH)
        def _start_next_gather():
            i_copy(jj + 1, (r + 1) % UNROLL).wait()
            g_copy(jj + 1, (r + 1) % UNROLL, 1 - b).start()

        @pl.when(jj + 2 < NCH)
        def _start_next_idx():
            i_copy(jj + 2, (r + 2) % UNROLL).start()
            c_copy(jj + 2, (r + 2) % UNROLL).start()

        g_copy(jj, r, b).wait()
        c_copy(jj, r).wait()
        rowsb = rows[b]

        def edge_group(g, carry):
            c0g = cbuf[r, 0, pl.ds(g * L, L)]
            c1g = cbuf[r, 1, pl.ds(g * L, L)]
            for k in range(L):
                e = g * L + k
                s0 = _splat(c0g, k)
                s1 = _splat(c1g, k)
                for h in range(OUT // L):
                    a = rowsb[e, pl.ds(h * L, L)]
                    b2 = rowsb[e, pl.ds(OUT + h * L, L)]
                    msgv[e, pl.ds(h * L, L)] = s0 * a + s1 * b2
            return carry
        lax.fori_loop(0, CH // L, edge_group, 0)

        pltpu.sync_copy(msgv, accum.at[ibuf.at[r, 1]], add=True)

    i_copy(0, 0).start()
    c_copy(0, 0).start()
    i_copy(0, 0).wait()
    g_copy(0, 0, 0).start()
    i_copy(1, 1).start()
    c_copy(1, 1).start()

    def outer(jm, carry):
        j0 = UNROLL * jm
        for u in range(UNROLL):
            chunk(j0 + u, u, u % 2)
        return carry
    lax.fori_loop(0, NCH // UNROLL, outer, 0)

    plsc.subcore_barrier()
    pltpu.sync_copy(accum.at[pl.ds(base_row, ROWS_PT)],
                    out_hbm.at[cid, pl.ds(base_row, ROWS_PT)])

    @pl.when(sid == NS - 1)
    def _copy_tail():
        pltpu.sync_copy(accum.at[pl.ds(NS * ROWS_PT, REM_ROWS)],
                        out_hbm.at[cid, pl.ds(NS * ROWS_PT, REM_ROWS)])


def _sc_call(y, idxp, cfp):
    mesh = plsc.VectorSubcoreMesh(core_axis_name="c", subcore_axis_name="s")
    fn = pl.kernel(
        _sc_edges,
        out_type=jax.ShapeDtypeStruct((NC, N, OUT), jnp.float32),
        mesh=mesh,
        scratch_types=[
            pltpu.VMEM((UNROLL, 2, CH), jnp.int32),
            pltpu.VMEM((UNROLL, 2, CH), jnp.float32),
            pltpu.VMEM((CH, NB * OUT), jnp.float32),
            pltpu.VMEM((CH, NB * OUT), jnp.float32),
            pltpu.VMEM((CH, OUT), jnp.float32),
            pltpu.VMEM_SHARED((N, OUT), jnp.float32),
        ] + [pltpu.SemaphoreType.DMA] * 10,
    )
    return fn(y, idxp, cfp)


def kernel(x, edge_index, edge_attr, basis, att, root, bias):
    pad = E_PAD - E
    src = jnp.pad(edge_index[0], (0, pad)).reshape(NW, NCH, CH)
    dst = jnp.pad(edge_index[1], (0, pad)).reshape(NW, NCH, CH)
    eat = jnp.pad(edge_attr, ((0, pad), (0, 0))).T

    y, base, c0, c1 = pl.pallas_call(
        _tc_prep,
        out_shape=[
            jax.ShapeDtypeStruct((N, NB * OUT), jnp.float32),
            jax.ShapeDtypeStruct((N, OUT), jnp.float32),
            jax.ShapeDtypeStruct((E_PAD,), jnp.float32),
            jax.ShapeDtypeStruct((E_PAD,), jnp.float32),
        ],
    )(x, basis, root, bias, eat, att)

    idxp = jnp.stack([src, dst], axis=2)
    cfp = jnp.stack([c0.reshape(NW, NCH, CH),
                     c1.reshape(NW, NCH, CH)], axis=2)

    partials = _sc_call(y, idxp, cfp)

    out = pl.pallas_call(
        _tc_final,
        out_shape=jax.ShapeDtypeStruct((N, OUT), jnp.float32),
    )(base, partials)
    return out
